# baseline (speedup 1.0000x reference)
import jax
import jax.numpy as jnp
from jax import lax
from jax.experimental import pallas as pl
from jax.experimental.pallas import tpu as pltpu
from jax.experimental.pallas import tpu_sc as plsc

_NUM_CLASSES = 128
_N_SUPPORT = 160000
_N_QUERY = 4096
_D_IN = 128
_D_EMB = 64

_BLK = 256
_NBLK = _N_SUPPORT // _BLK
_NW = 32
_K_MAX = -(-_NBLK // _NW)


def _sc_body(x_hbm, lbl_hbm, sums_hbm, cnt_hbm,
             buf0, buf1, la0, lb0, la1, lb1, zer, cntv, stage_v,
             sums_acc, cnt_stage, sem0, sem1):
    cid = lax.axis_index("c")
    sid = lax.axis_index("s")
    wid = sid * 2 + cid

    zero16 = jnp.zeros((16,), jnp.float32)
    one16 = jnp.ones((16,), jnp.float32)

    def _fill_zer(i, c):
        for j in range(_D_IN // 16):
            zer[i, pl.ds(j * 16, 16)] = zero16
        return c

    lax.fori_loop(0, 8, _fill_zer, 0)
    for j in range(_NUM_CLASSES // 16):
        cntv[pl.ds(j * 16, 16)] = zero16

    pltpu.sync_copy(zer, sums_acc.at[pl.ds(sid * 8, 8)])
    plsc.subcore_barrier()

    bufs = (buf0, buf1)
    lbls = ((la0, lb0), (la1, lb1))
    sems = (sem0, sem1)

    def _issue(k, slot):
        blk = k * _NW + wid

        @pl.when(blk < _NBLK)
        def _():
            base = blk * _BLK
            pltpu.async_copy(x_hbm.at[pl.ds(base, _BLK)], bufs[slot], sems[slot])
            pltpu.async_copy(lbl_hbm.at[pl.ds(base, 128)], lbls[slot][0], sems[slot])
            pltpu.async_copy(lbl_hbm.at[pl.ds(base + 128, 128)], lbls[slot][1], sems[slot])

    def _drain_scatter(k, slot):
        blk = k * _NW + wid

        @pl.when(blk < _NBLK)
        def _():
            base = blk * _BLK
            pltpu.make_async_copy(
                x_hbm.at[pl.ds(base, _BLK)], bufs[slot], sems[slot]).wait()
            pltpu.make_async_copy(
                lbl_hbm.at[pl.ds(base, 128)], lbls[slot][0], sems[slot]).wait()
            pltpu.make_async_copy(
                lbl_hbm.at[pl.ds(base + 128, 128)], lbls[slot][1], sems[slot]).wait()
            pltpu.sync_copy(bufs[slot].at[pl.ds(0, 128)],
                            sums_acc.at[lbls[slot][0]], add=True)
            pltpu.sync_copy(bufs[slot].at[pl.ds(128, 128)],
                            sums_acc.at[lbls[slot][1]], add=True)
            for p in range(2):
                for j in range(128 // 16):
                    lv = lbls[slot][p][pl.ds(j * 16, 16)]
                    plsc.addupdate_scatter(cntv, [lv], one16)

    for slot in range(2):
        _issue(slot, slot)

    def _step(kk, c):
        for slot in range(2):
            k = kk * 2 + slot
            _drain_scatter(k, slot)
            _issue(k + 2, slot)
        return c

    lax.fori_loop(0, _K_MAX // 2, _step, 0)
    pltpu.sync_copy(cntv, cnt_stage.at[sid])
    plsc.subcore_barrier()

    @pl.when(sid == 0)
    def _():
        pltpu.sync_copy(sums_acc, sums_hbm.at[cid])
        pltpu.sync_copy(cnt_stage, stage_v)
        for j in range(_NUM_CLASSES // 16):
            acc = stage_v[0, pl.ds(j * 16, 16)]
            for r in range(1, 16):
                acc = acc + stage_v[r, pl.ds(j * 16, 16)]
            cntv[pl.ds(j * 16, 16)] = acc
        pltpu.sync_copy(cntv, cnt_hbm.at[cid])


def _make_sc_call():
    mesh = plsc.VectorSubcoreMesh(core_axis_name="c", subcore_axis_name="s")
    return pl.kernel(
        _sc_body,
        out_type=(
            jax.ShapeDtypeStruct((2, _NUM_CLASSES, _D_IN), jnp.float32),
            jax.ShapeDtypeStruct((2, _NUM_CLASSES), jnp.float32),
        ),
        mesh=mesh,
        compiler_params=pltpu.CompilerParams(needs_layout_passes=False),
        scratch_types=[
            pltpu.VMEM((_BLK, _D_IN), jnp.float32),
            pltpu.VMEM((_BLK, _D_IN), jnp.float32),
            pltpu.VMEM((128,), jnp.int32),
            pltpu.VMEM((128,), jnp.int32),
            pltpu.VMEM((128,), jnp.int32),
            pltpu.VMEM((128,), jnp.int32),
            pltpu.VMEM((8, _D_IN), jnp.float32),
            pltpu.VMEM((_NUM_CLASSES,), jnp.float32),
            pltpu.VMEM((16, _NUM_CLASSES), jnp.float32),
            pltpu.VMEM_SHARED((_NUM_CLASSES, _D_IN), jnp.float32),
            pltpu.VMEM_SHARED((16, _NUM_CLASSES), jnp.float32),
            pltpu.SemaphoreType.DMA,
            pltpu.SemaphoreType.DMA,
        ],
    )


def _tc_body(sums_ref, cnt_ref, w_ref, b_ref, q_ref, prob_ref, dist_ref):
    S = sums_ref[0] + sums_ref[1]
    crow = cnt_ref[0:1, :] + cnt_ref[1:2, :]
    r_i = lax.broadcasted_iota(jnp.int32, (_NUM_CLASSES, _NUM_CLASSES), 0)
    c_i = lax.broadcasted_iota(jnp.int32, (_NUM_CLASSES, _NUM_CLASSES), 1)
    diag = jnp.where(r_i == c_i, jnp.broadcast_to(crow, r_i.shape), 0.0)
    cntcol = jnp.sum(diag, axis=1, keepdims=True)
    W = w_ref[...]
    b = b_ref[...]
    SW = jnp.dot(S, W, preferred_element_type=jnp.float32)
    proto = (SW + cntcol * b) / jnp.maximum(cntcol, 1.0)

    qe = jnp.dot(q_ref[...], W, preferred_element_type=jnp.float32) + b
    qn = jnp.sum(qe * qe, axis=1, keepdims=True)
    pn = jnp.sum(proto * proto, axis=1)[None, :]
    cross = jnp.dot(qe, proto.T, preferred_element_type=jnp.float32)
    d = qn + pn - 2.0 * cross
    dist_ref[...] = d
    nd = -d
    m = jnp.max(nd, axis=1, keepdims=True)
    e = jnp.exp(nd - m)
    prob_ref[...] = e / jnp.sum(e, axis=1, keepdims=True)


_BQ = 512


def _tc_call(sums, cnt, W, b2, query_set, interpret=False):
    grid = (_N_QUERY // _BQ,)
    return pl.pallas_call(
        _tc_body,
        grid=grid,
        in_specs=[
            pl.BlockSpec((2, _NUM_CLASSES, _D_IN), lambda i: (0, 0, 0)),
            pl.BlockSpec((2, _NUM_CLASSES), lambda i: (0, 0)),
            pl.BlockSpec((_D_IN, _D_EMB), lambda i: (0, 0)),
            pl.BlockSpec((1, _D_EMB), lambda i: (0, 0)),
            pl.BlockSpec((_BQ, _D_IN), lambda i: (i, 0)),
        ],
        out_specs=[
            pl.BlockSpec((_BQ, _NUM_CLASSES), lambda i: (i, 0)),
            pl.BlockSpec((_BQ, _NUM_CLASSES), lambda i: (i, 0)),
        ],
        out_shape=[
            jax.ShapeDtypeStruct((_N_QUERY, _NUM_CLASSES), jnp.float32),
            jax.ShapeDtypeStruct((_N_QUERY, _NUM_CLASSES), jnp.float32),
        ],
        interpret=interpret,
    )(sums, cnt, W, b2, query_set)


def kernel(support_set, support_labels, query_set, W, b):
    labels = support_labels.astype(jnp.int32)
    sums, cnt = _make_sc_call()(support_set, labels)
    prob, dist = _tc_call(sums, cnt, W, b.reshape(1, _D_EMB), query_set)
    class_labels = jnp.arange(_NUM_CLASSES, dtype=support_labels.dtype)
    return (prob, class_labels, dist)


# speedup vs baseline: 1.0344x; 1.0344x over previous
import jax
import jax.numpy as jnp
from jax import lax
from jax.experimental import pallas as pl
from jax.experimental.pallas import tpu as pltpu
from jax.experimental.pallas import tpu_sc as plsc

_NUM_CLASSES = 128
_N_SUPPORT = 160000
_N_QUERY = 4096
_D_IN = 128
_D_EMB = 64

_BLK = 256
_NBLK = _N_SUPPORT // _BLK
_NW = 32
_K_MAX = -(-_NBLK // _NW)


def _sc_body(x_hbm, lbl_hbm, sums_hbm, cnt_hbm,
             buf0, buf1, la0, lb0, la1, lb1, zer, cntv, stage_v,
             sums_acc, cnt_stage, sem0, sem1):
    cid = lax.axis_index("c")
    sid = lax.axis_index("s")
    wid = sid * 2 + cid

    zero16 = jnp.zeros((16,), jnp.float32)
    one16 = jnp.ones((16,), jnp.float32)

    def _fill_zer(i, c):
        for j in range(_D_IN // 16):
            zer[i, pl.ds(j * 16, 16)] = zero16
        return c

    lax.fori_loop(0, 8, _fill_zer, 0)
    for j in range(_NUM_CLASSES // 16):
        cntv[pl.ds(j * 16, 16)] = zero16

    pltpu.sync_copy(zer, sums_acc.at[pl.ds(sid * 8, 8)])
    plsc.subcore_barrier()

    bufs = (buf0, buf1)
    lbls = ((la0, lb0), (la1, lb1))
    sems = (sem0, sem1)

    def _issue(k, slot):
        blk = k * _NW + wid

        @pl.when(blk < _NBLK)
        def _():
            base = blk * _BLK
            pltpu.async_copy(x_hbm.at[pl.ds(base, _BLK)], bufs[slot], sems[slot])
            pltpu.async_copy(lbl_hbm.at[pl.ds(base, 128)], lbls[slot][0], sems[slot])
            pltpu.async_copy(lbl_hbm.at[pl.ds(base + 128, 128)], lbls[slot][1], sems[slot])

    def _drain_scatter(k, slot):
        blk = k * _NW + wid

        @pl.when(blk < _NBLK)
        def _():
            base = blk * _BLK
            pltpu.make_async_copy(
                x_hbm.at[pl.ds(base, _BLK)], bufs[slot], sems[slot]).wait()
            pltpu.make_async_copy(
                lbl_hbm.at[pl.ds(base, 128)], lbls[slot][0], sems[slot]).wait()
            pltpu.make_async_copy(
                lbl_hbm.at[pl.ds(base + 128, 128)], lbls[slot][1], sems[slot]).wait()
            pltpu.sync_copy(bufs[slot].at[pl.ds(0, 128)],
                            sums_acc.at[lbls[slot][0]], add=True)
            pltpu.sync_copy(bufs[slot].at[pl.ds(128, 128)],
                            sums_acc.at[lbls[slot][1]], add=True)
            for p in range(2):
                for j in range(128 // 16):
                    lv = lbls[slot][p][pl.ds(j * 16, 16)]
                    plsc.addupdate_scatter(cntv, [lv], one16)

    for slot in range(2):
        _issue(slot, slot)

    def _step(kk, c):
        for slot in range(2):
            k = kk * 2 + slot
            _drain_scatter(k, slot)
            _issue(k + 2, slot)
        return c

    lax.fori_loop(0, _K_MAX // 2, _step, 0)
    pltpu.sync_copy(cntv, cnt_stage.at[sid])
    plsc.subcore_barrier()

    @pl.when(sid == 0)
    def _():
        pltpu.sync_copy(sums_acc, sums_hbm.at[cid])
        pltpu.sync_copy(cnt_stage, stage_v)
        for j in range(_NUM_CLASSES // 16):
            acc = stage_v[0, pl.ds(j * 16, 16)]
            for r in range(1, 16):
                acc = acc + stage_v[r, pl.ds(j * 16, 16)]
            cntv[pl.ds(j * 16, 16)] = acc
        pltpu.sync_copy(cntv, cnt_hbm.at[cid])


def _make_sc_call():
    mesh = plsc.VectorSubcoreMesh(core_axis_name="c", subcore_axis_name="s")
    return pl.kernel(
        _sc_body,
        out_type=(
            jax.ShapeDtypeStruct((2, _NUM_CLASSES, _D_IN), jnp.float32),
            jax.ShapeDtypeStruct((2, _NUM_CLASSES), jnp.float32),
        ),
        mesh=mesh,
        compiler_params=pltpu.CompilerParams(needs_layout_passes=False),
        scratch_types=[
            pltpu.VMEM((_BLK, _D_IN), jnp.float32),
            pltpu.VMEM((_BLK, _D_IN), jnp.float32),
            pltpu.VMEM((128,), jnp.int32),
            pltpu.VMEM((128,), jnp.int32),
            pltpu.VMEM((128,), jnp.int32),
            pltpu.VMEM((128,), jnp.int32),
            pltpu.VMEM((8, _D_IN), jnp.float32),
            pltpu.VMEM((_NUM_CLASSES,), jnp.float32),
            pltpu.VMEM((16, _NUM_CLASSES), jnp.float32),
            pltpu.VMEM_SHARED((_NUM_CLASSES, _D_IN), jnp.float32),
            pltpu.VMEM_SHARED((16, _NUM_CLASSES), jnp.float32),
            pltpu.SemaphoreType.DMA,
            pltpu.SemaphoreType.DMA,
        ],
    )


_BQ = 1024


def _qe_body(q_ref, w_ref, b_ref, qe_ref):
    qe_ref[...] = (jnp.dot(q_ref[...], w_ref[...],
                           preferred_element_type=jnp.float32) + b_ref[...])


def _qe_call(query_set, W, b2, interpret=False):
    return pl.pallas_call(
        _qe_body,
        grid=(_N_QUERY // _BQ,),
        in_specs=[
            pl.BlockSpec((_BQ, _D_IN), lambda i: (i, 0)),
            pl.BlockSpec((_D_IN, _D_EMB), lambda i: (0, 0)),
            pl.BlockSpec((1, _D_EMB), lambda i: (0, 0)),
        ],
        out_specs=pl.BlockSpec((_BQ, _D_EMB), lambda i: (i, 0)),
        out_shape=jax.ShapeDtypeStruct((_N_QUERY, _D_EMB), jnp.float32),
        interpret=interpret,
    )(query_set, W, b2)


def _tc_body(sums_ref, cnt_ref, w_ref, b_ref, qe_ref, prob_ref, dist_ref):
    S = sums_ref[0] + sums_ref[1]
    crow = cnt_ref[0:1, :] + cnt_ref[1:2, :]
    r_i = lax.broadcasted_iota(jnp.int32, (_NUM_CLASSES, _NUM_CLASSES), 0)
    c_i = lax.broadcasted_iota(jnp.int32, (_NUM_CLASSES, _NUM_CLASSES), 1)
    diag = jnp.where(r_i == c_i, jnp.broadcast_to(crow, r_i.shape), 0.0)
    cntcol = jnp.sum(diag, axis=1, keepdims=True)
    W = w_ref[...]
    b = b_ref[...]
    SW = jnp.dot(S, W, preferred_element_type=jnp.float32)
    proto = (SW + cntcol * b) / jnp.maximum(cntcol, 1.0)

    qe = qe_ref[...]
    qn = jnp.sum(qe * qe, axis=1, keepdims=True)
    pn = jnp.sum(proto * proto, axis=1)[None, :]
    cross = jnp.dot(qe, proto.T, preferred_element_type=jnp.float32)
    d = qn + pn - 2.0 * cross
    dist_ref[...] = d
    nd = -d
    m = jnp.max(nd, axis=1, keepdims=True)
    e = jnp.exp(nd - m)
    prob_ref[...] = e / jnp.sum(e, axis=1, keepdims=True)


def _tc_call(sums, cnt, W, b2, qe, interpret=False):
    grid = (_N_QUERY // _BQ,)
    return pl.pallas_call(
        _tc_body,
        grid=grid,
        in_specs=[
            pl.BlockSpec((2, _NUM_CLASSES, _D_IN), lambda i: (0, 0, 0)),
            pl.BlockSpec((2, _NUM_CLASSES), lambda i: (0, 0)),
            pl.BlockSpec((_D_IN, _D_EMB), lambda i: (0, 0)),
            pl.BlockSpec((1, _D_EMB), lambda i: (0, 0)),
            pl.BlockSpec((_BQ, _D_EMB), lambda i: (i, 0)),
        ],
        out_specs=[
            pl.BlockSpec((_BQ, _NUM_CLASSES), lambda i: (i, 0)),
            pl.BlockSpec((_BQ, _NUM_CLASSES), lambda i: (i, 0)),
        ],
        out_shape=[
            jax.ShapeDtypeStruct((_N_QUERY, _NUM_CLASSES), jnp.float32),
            jax.ShapeDtypeStruct((_N_QUERY, _NUM_CLASSES), jnp.float32),
        ],
        interpret=interpret,
    )(sums, cnt, W, b2, qe)


def kernel(support_set, support_labels, query_set, W, b):
    labels = support_labels.astype(jnp.int32)
    b2 = b.reshape(1, _D_EMB)
    qe = _qe_call(query_set, W, b2)
    sums, cnt = _make_sc_call()(support_set, labels)
    prob, dist = _tc_call(sums, cnt, W, b2, qe)
    class_labels = jnp.arange(_NUM_CLASSES, dtype=support_labels.dtype)
    return (prob, class_labels, dist)


# speedup vs baseline: 1.2272x; 1.1864x over previous
import jax
import jax.numpy as jnp
from jax import lax
from jax.experimental import pallas as pl
from jax.experimental.pallas import tpu as pltpu
from jax.experimental.pallas import tpu_sc as plsc

_NUM_CLASSES = 128
_N_SUPPORT = 160000
_N_QUERY = 4096
_D_IN = 128
_D_EMB = 64

_BLK = 256
_SC_ROWS = 102400
_TC_ROWS = _N_SUPPORT - _SC_ROWS
_TCB = 1600
_NBLK = _SC_ROWS // _BLK
_NW = 32
_K_MAX = -(-_NBLK // _NW)


def _sc_body(x_hbm, lbl_hbm, sums_hbm, cnt_hbm,
             buf0, buf1, la0, lb0, la1, lb1, zer, cntv, stage_v,
             sums_acc, cnt_stage, sem0, sem1):
    cid = lax.axis_index("c")
    sid = lax.axis_index("s")
    wid = sid * 2 + cid

    zero16 = jnp.zeros((16,), jnp.float32)
    one16 = jnp.ones((16,), jnp.float32)

    def _fill_zer(i, c):
        for j in range(_D_IN // 16):
            zer[i, pl.ds(j * 16, 16)] = zero16
        return c

    lax.fori_loop(0, 8, _fill_zer, 0)
    for j in range(_NUM_CLASSES // 16):
        cntv[pl.ds(j * 16, 16)] = zero16

    pltpu.sync_copy(zer, sums_acc.at[pl.ds(sid * 8, 8)])
    plsc.subcore_barrier()

    bufs = (buf0, buf1)
    lbls = ((la0, lb0), (la1, lb1))
    sems = (sem0, sem1)

    def _issue(k, slot):
        blk = k * _NW + wid

        @pl.when(blk < _NBLK)
        def _():
            base = blk * _BLK
            pltpu.async_copy(x_hbm.at[pl.ds(base, _BLK)], bufs[slot], sems[slot])
            pltpu.async_copy(lbl_hbm.at[pl.ds(base, 128)], lbls[slot][0], sems[slot])
            pltpu.async_copy(lbl_hbm.at[pl.ds(base + 128, 128)], lbls[slot][1], sems[slot])

    def _drain_scatter(k, slot):
        blk = k * _NW + wid

        @pl.when(blk < _NBLK)
        def _():
            base = blk * _BLK
            pltpu.make_async_copy(
                x_hbm.at[pl.ds(base, _BLK)], bufs[slot], sems[slot]).wait()
            pltpu.make_async_copy(
                lbl_hbm.at[pl.ds(base, 128)], lbls[slot][0], sems[slot]).wait()
            pltpu.make_async_copy(
                lbl_hbm.at[pl.ds(base + 128, 128)], lbls[slot][1], sems[slot]).wait()
            pltpu.sync_copy(bufs[slot].at[pl.ds(0, 128)],
                            sums_acc.at[lbls[slot][0]], add=True)
            pltpu.sync_copy(bufs[slot].at[pl.ds(128, 128)],
                            sums_acc.at[lbls[slot][1]], add=True)
            for p in range(2):
                for j in range(128 // 16):
                    lv = lbls[slot][p][pl.ds(j * 16, 16)]
                    plsc.addupdate_scatter(cntv, [lv], one16)

    for slot in range(2):
        _issue(slot, slot)

    def _step(kk, c):
        for slot in range(2):
            k = kk * 2 + slot
            _drain_scatter(k, slot)
            _issue(k + 2, slot)
        return c

    lax.fori_loop(0, -(-_K_MAX // 2), _step, 0)
    pltpu.sync_copy(cntv, cnt_stage.at[sid])
    plsc.subcore_barrier()

    @pl.when(sid == 0)
    def _():
        pltpu.sync_copy(sums_acc, sums_hbm.at[cid])
        pltpu.sync_copy(cnt_stage, stage_v)
        for j in range(_NUM_CLASSES // 16):
            acc = stage_v[0, pl.ds(j * 16, 16)]
            for r in range(1, 16):
                acc = acc + stage_v[r, pl.ds(j * 16, 16)]
            cntv[pl.ds(j * 16, 16)] = acc
        pltpu.sync_copy(cntv, cnt_hbm.at[cid])


def _make_sc_call():
    mesh = plsc.VectorSubcoreMesh(core_axis_name="c", subcore_axis_name="s")
    return pl.kernel(
        _sc_body,
        out_type=(
            jax.ShapeDtypeStruct((2, _NUM_CLASSES, _D_IN), jnp.float32),
            jax.ShapeDtypeStruct((2, _NUM_CLASSES), jnp.float32),
        ),
        mesh=mesh,
        compiler_params=pltpu.CompilerParams(needs_layout_passes=False),
        scratch_types=[
            pltpu.VMEM((_BLK, _D_IN), jnp.float32),
            pltpu.VMEM((_BLK, _D_IN), jnp.float32),
            pltpu.VMEM((128,), jnp.int32),
            pltpu.VMEM((128,), jnp.int32),
            pltpu.VMEM((128,), jnp.int32),
            pltpu.VMEM((128,), jnp.int32),
            pltpu.VMEM((8, _D_IN), jnp.float32),
            pltpu.VMEM((_NUM_CLASSES,), jnp.float32),
            pltpu.VMEM((16, _NUM_CLASSES), jnp.float32),
            pltpu.VMEM_SHARED((_NUM_CLASSES, _D_IN), jnp.float32),
            pltpu.VMEM_SHARED((16, _NUM_CLASSES), jnp.float32),
            pltpu.SemaphoreType.DMA,
            pltpu.SemaphoreType.DMA,
        ],
    )


def _seg_body(lab_ref, x_ref, tsum_ref, tcnt_ref):
    i = pl.program_id(0)
    lab = lab_ref[0]
    iota = lax.broadcasted_iota(jnp.int32, (_NUM_CLASSES, _TCB), 0)
    oh = jnp.where(lab == iota, 1.0, 0.0)
    psum = jnp.dot(oh, x_ref[...], preferred_element_type=jnp.float32)
    pcnt = jnp.dot(oh, jnp.ones((_TCB, 8), jnp.float32),
                   preferred_element_type=jnp.float32)

    @pl.when(i == 0)
    def _():
        tsum_ref[...] = psum
        tcnt_ref[...] = pcnt

    @pl.when(i > 0)
    def _():
        tsum_ref[...] += psum
        tcnt_ref[...] += pcnt


def _seg_call(lab_tc, support_set, interpret=False):
    nsteps = _TC_ROWS // _TCB
    blk0 = _SC_ROWS // _TCB
    return pl.pallas_call(
        _seg_body,
        grid=(nsteps,),
        in_specs=[
            pl.BlockSpec((1, 1, _TCB), lambda i: (i, 0, 0)),
            pl.BlockSpec((_TCB, _D_IN), lambda i: (i + blk0, 0)),
        ],
        out_specs=[
            pl.BlockSpec((_NUM_CLASSES, _D_IN), lambda i: (0, 0)),
            pl.BlockSpec((_NUM_CLASSES, 8), lambda i: (0, 0)),
        ],
        out_shape=[
            jax.ShapeDtypeStruct((_NUM_CLASSES, _D_IN), jnp.float32),
            jax.ShapeDtypeStruct((_NUM_CLASSES, 8), jnp.float32),
        ],
        interpret=interpret,
    )(lab_tc, support_set)


_BQ = 1024


def _qe_body(q_ref, w_ref, b_ref, qe_ref):
    qe_ref[...] = (jnp.dot(q_ref[...], w_ref[...],
                           preferred_element_type=jnp.float32) + b_ref[...])


def _qe_call(query_set, W, b2, interpret=False):
    return pl.pallas_call(
        _qe_body,
        grid=(_N_QUERY // _BQ,),
        in_specs=[
            pl.BlockSpec((_BQ, _D_IN), lambda i: (i, 0)),
            pl.BlockSpec((_D_IN, _D_EMB), lambda i: (0, 0)),
            pl.BlockSpec((1, _D_EMB), lambda i: (0, 0)),
        ],
        out_specs=pl.BlockSpec((_BQ, _D_EMB), lambda i: (i, 0)),
        out_shape=jax.ShapeDtypeStruct((_N_QUERY, _D_EMB), jnp.float32),
        interpret=interpret,
    )(query_set, W, b2)


def _tc_body(sums_ref, cnt_ref, tsum_ref, tcnt_ref, w_ref, b_ref, qe_ref,
             prob_ref, dist_ref):
    S = sums_ref[0] + sums_ref[1] + tsum_ref[...]
    crow = cnt_ref[0:1, :] + cnt_ref[1:2, :]
    r_i = lax.broadcasted_iota(jnp.int32, (_NUM_CLASSES, _NUM_CLASSES), 0)
    c_i = lax.broadcasted_iota(jnp.int32, (_NUM_CLASSES, _NUM_CLASSES), 1)
    diag = jnp.where(r_i == c_i, jnp.broadcast_to(crow, r_i.shape), 0.0)
    cntcol = jnp.sum(diag, axis=1, keepdims=True) + tcnt_ref[:, 0:1]
    W = w_ref[...]
    b = b_ref[...]
    SW = jnp.dot(S, W, preferred_element_type=jnp.float32)
    proto = (SW + cntcol * b) / jnp.maximum(cntcol, 1.0)

    qe = qe_ref[...]
    qn = jnp.sum(qe * qe, axis=1, keepdims=True)
    pn = jnp.sum(proto * proto, axis=1)[None, :]
    cross = jnp.dot(qe, proto.T, preferred_element_type=jnp.float32)
    d = qn + pn - 2.0 * cross
    dist_ref[...] = d
    nd = -d
    m = jnp.max(nd, axis=1, keepdims=True)
    e = jnp.exp(nd - m)
    prob_ref[...] = e / jnp.sum(e, axis=1, keepdims=True)


def _tc_call(sums, cnt, tsum, tcnt, W, b2, qe, interpret=False):
    grid = (_N_QUERY // _BQ,)
    return pl.pallas_call(
        _tc_body,
        grid=grid,
        in_specs=[
            pl.BlockSpec((2, _NUM_CLASSES, _D_IN), lambda i: (0, 0, 0)),
            pl.BlockSpec((2, _NUM_CLASSES), lambda i: (0, 0)),
            pl.BlockSpec((_NUM_CLASSES, _D_IN), lambda i: (0, 0)),
            pl.BlockSpec((_NUM_CLASSES, 8), lambda i: (0, 0)),
            pl.BlockSpec((_D_IN, _D_EMB), lambda i: (0, 0)),
            pl.BlockSpec((1, _D_EMB), lambda i: (0, 0)),
            pl.BlockSpec((_BQ, _D_EMB), lambda i: (i, 0)),
        ],
        out_specs=[
            pl.BlockSpec((_BQ, _NUM_CLASSES), lambda i: (i, 0)),
            pl.BlockSpec((_BQ, _NUM_CLASSES), lambda i: (i, 0)),
        ],
        out_shape=[
            jax.ShapeDtypeStruct((_N_QUERY, _NUM_CLASSES), jnp.float32),
            jax.ShapeDtypeStruct((_N_QUERY, _NUM_CLASSES), jnp.float32),
        ],
        interpret=interpret,
    )(sums, cnt, tsum, tcnt, W, b2, qe)


def kernel(support_set, support_labels, query_set, W, b):
    labels = support_labels.astype(jnp.int32)
    b2 = b.reshape(1, _D_EMB)
    lab_tc = labels[_SC_ROWS:].reshape(_TC_ROWS // _TCB, 1, _TCB)
    qe = _qe_call(query_set, W, b2)
    tsum, tcnt = _seg_call(lab_tc, support_set)
    sums, cnt = _make_sc_call()(support_set, labels)
    prob, dist = _tc_call(sums, cnt, tsum, tcnt, W, b2, qe)
    class_labels = jnp.arange(_NUM_CLASSES, dtype=support_labels.dtype)
    return (prob, class_labels, dist)
